# Initial kernel scaffold; baseline (speedup 1.0000x reference)
import jax
import jax.numpy as jnp
from jax.experimental import pallas as pl


def kernel(x, edge_index, weight, W1, b1, W2, b2, W3, b3, lin1_W, lin1_b, Wc, bc, lin2_W, lin2_b):
    raise NotImplementedError("write your pallas kernel here")


# speedup vs baseline: 19.8840x; 19.8840x over previous
import dataclasses
import functools

import jax
import jax.numpy as jnp
from jax import lax
from jax.experimental import pallas as pl
from jax.experimental.pallas import tpu as pltpu
from jax.experimental.pallas import tpu_sc as plsc

NN = 10000
EE = 320000
DD = 128
HH = 16

NC = 2
NS = 16
NW = NC * NS
NPAD = 10240
SLICE = NPAD // NS
EPW = 10240
EPAD = NW * EPW
CH = 1024
NCHK = EPW // CH
RPC = CH // 128
ROWS_PW = EPW // 128

@functools.cache
def _sc_mesh():
    return plsc.VectorSubcoreMesh(core_axis_name="c", subcore_axis_name="s",
                                  num_cores=NC, num_subcores=NS)


def _sc_params():
    cp = pltpu.CompilerParams()
    cp = dataclasses.replace(cp, needs_layout_passes=False,
                             use_tc_tiling_on_sc=False)
    return cp


@functools.cache
def _sc_layer_kernel():
    return functools.partial(
        pl.kernel,
        out_type=jax.ShapeDtypeStruct((NC, NPAD, HH), jnp.float32),
        mesh=_sc_mesh(),
        compiler_params=_sc_params(),
        scratch_types=[
            pltpu.VMEM((RPC, 128), jnp.int32),
            pltpu.VMEM((RPC, 128), jnp.int32),
            pltpu.VMEM((CH,), jnp.float32),
            pltpu.VMEM((CH, HH), jnp.float32),
            pltpu.VMEM_SHARED((NPAD, HH), jnp.float32),
            pltpu.SemaphoreType.DMA,
        ],
    )(_sc_layer_body)


def _sc_layer(src2d, dst2d, wp, g, z):
    return _sc_layer_kernel()(src2d, dst2d, wp, g, z)


def _sc_layer_body(src_hbm, dst_hbm, w_hbm, g_hbm, z_hbm, out_hbm,
                   si_v, di_v, w_v, rows_v, acc_sh, sem):
    c = lax.axis_index("c")
    s = lax.axis_index("s")
    wid = s * NC + c
    pltpu.sync_copy(z_hbm.at[pl.ds(s * SLICE, SLICE)],
                    acc_sh.at[pl.ds(s * SLICE, SLICE)])
    plsc.subcore_barrier()

    base_row = wid * ROWS_PW
    base_e = wid * EPW

    @pl.loop(0, NCHK)
    def _chunk(t):
        r0 = base_row + t * RPC
        e0 = base_e + t * CH
        pltpu.sync_copy(src_hbm.at[pl.ds(r0, RPC)], si_v)
        pltpu.sync_copy(dst_hbm.at[pl.ds(r0, RPC)], di_v)
        pltpu.sync_copy(w_hbm.at[pl.ds(e0, CH)], w_v)
        cps = [
            pltpu.async_copy(g_hbm.at[si_v.at[j]],
                             rows_v.at[pl.ds(j * 128, 128)], sem)
            for j in range(RPC)
        ]
        for cp in cps:
            cp.wait()

        @pl.loop(0, CH)
        def _scale(e):
            spl = plsc.load_gather(w_v, [jnp.full((16,), e, jnp.int32)])
            rows_v[e, :] = rows_v[e, :] * spl

        for j in range(RPC):
            pltpu.sync_copy(rows_v.at[pl.ds(j * 128, 128)],
                            acc_sh.at[di_v.at[j]], add=True)

    plsc.subcore_barrier()
    pltpu.sync_copy(acc_sh.at[pl.ds(s * SLICE, SLICE)],
                    out_hbm.at[c, pl.ds(s * SLICE, SLICE)])


def _scalar_accumulate_and_reduce(body_per_group, out_hbm,
                                  d_v, w_v, acc_v, red_v, out_v, red_sh,
                                  dst_hbm, w_hbm, load_extra):
    c = lax.axis_index("c")
    s = lax.axis_index("s")
    wid = s * NC + c

    @pl.loop(0, NPAD // 16)
    def _zero(i):
        acc_v[pl.ds(i * 16, 16)] = jnp.zeros((16,), jnp.float32)

    @pl.loop(0, NCHK)
    def _chunk(t):
        e0 = wid * EPW + t * CH
        pltpu.sync_copy(dst_hbm.at[pl.ds(e0, CH)], d_v)
        pltpu.sync_copy(w_hbm.at[pl.ds(e0, CH)], w_v)
        load_extra(e0)

        @pl.loop(0, CH // 16)
        def _grp(i):
            sl = pl.ds(i * 16, 16)
            plsc.addupdate_scatter(acc_v, [d_v[sl]], body_per_group(sl))

    pltpu.sync_copy(acc_v, red_sh.at[s])
    plsc.subcore_barrier()
    for r in range(NS):
        pltpu.sync_copy(red_sh.at[r, pl.ds(s * SLICE, SLICE)], red_v.at[r])

    @pl.loop(0, SLICE // 16)
    def _sum(i):
        sl = pl.ds(i * 16, 16)
        acc16 = red_v[0, sl]
        for r in range(1, NS):
            acc16 = acc16 + red_v[r, sl]
        out_v[sl] = acc16

    pltpu.sync_copy(out_v, out_hbm.at[c, pl.ds(s * SLICE, SLICE)])


_SCALAR_SCRATCH = [
    pltpu.VMEM((CH,), jnp.int32),
    pltpu.VMEM((CH,), jnp.float32),
    pltpu.VMEM((NPAD,), jnp.float32),
    pltpu.VMEM((NS, SLICE), jnp.float32),
    pltpu.VMEM((SLICE,), jnp.float32),
    pltpu.VMEM_SHARED((NS, NPAD), jnp.float32),
    pltpu.SemaphoreType.DMA,
]


def _sc_deg_body(dst_hbm, w_hbm, out_hbm,
                 d_v, w_v, acc_v, red_v, out_v, red_sh, sem):
    _scalar_accumulate_and_reduce(
        lambda sl: w_v[sl], out_hbm,
        d_v, w_v, acc_v, red_v, out_v, red_sh,
        dst_hbm, w_hbm, lambda e0: None)


@functools.cache
def _sc_deg_kernel():
    return functools.partial(
        pl.kernel,
        out_type=jax.ShapeDtypeStruct((NC, NPAD), jnp.float32),
        mesh=_sc_mesh(),
        compiler_params=_sc_params(),
        scratch_types=_SCALAR_SCRATCH,
    )(_sc_deg_body)


def _sc_deg(dstp, wp):
    return _sc_deg_kernel()(dstp, wp)


def _sc_conv1_body(src_hbm, dst_hbm, w_hbm, hh_hbm, out_hbm,
                   src_v, hh_v, d_v, w_v, acc_v, red_v, out_v, red_sh, sem):
    pltpu.sync_copy(hh_hbm, hh_v)

    def load_extra(e0):
        pltpu.sync_copy(src_hbm.at[pl.ds(e0, CH)], src_v)

    _scalar_accumulate_and_reduce(
        lambda sl: w_v[sl] * plsc.load_gather(hh_v, [src_v[sl]]), out_hbm,
        d_v, w_v, acc_v, red_v, out_v, red_sh,
        dst_hbm, w_hbm, load_extra)


@functools.cache
def _sc_conv1_kernel():
    return functools.partial(
        pl.kernel,
        out_type=jax.ShapeDtypeStruct((NC, NPAD), jnp.float32),
        mesh=_sc_mesh(),
        compiler_params=_sc_params(),
        scratch_types=[pltpu.VMEM((CH,), jnp.int32),
                       pltpu.VMEM((NPAD,), jnp.float32)] + _SCALAR_SCRATCH,
    )(_sc_conv1_body)


def _sc_conv1(srcp, dstp, wp, hh):
    return _sc_conv1_kernel()(srcp, dstp, wp, hh)

RB = 1000


def _dis_body(p_ref, o_ref):
    deg = p_ref[0] + p_ref[1] + 1.0
    o_ref[...] = jnp.where(deg > 0, lax.rsqrt(deg), 0.0)


def _tc_dis(degp):
    return pl.pallas_call(
        _dis_body,
        out_shape=jax.ShapeDtypeStruct((NPAD // 128, 128), jnp.float32),
    )(degp.reshape(NC, NPAD // 128, 128))


def _l1_body(x_ref, w_ref, dis_ref, o_ref):
    t = jnp.dot(x_ref[...], w_ref[...], preferred_element_type=jnp.float32)
    o_ref[...] = t * dis_ref[...]


def _tc_l1(x, W1, dis_col):
    return pl.pallas_call(
        _l1_body,
        grid=(NN // RB,),
        in_specs=[
            pl.BlockSpec((RB, DD), lambda i: (i, 0)),
            pl.BlockSpec((DD, HH), lambda i: (0, 0)),
            pl.BlockSpec((RB, 1), lambda i: (i, 0)),
        ],
        out_specs=pl.BlockSpec((RB, HH), lambda i: (i, 0)),
        out_shape=jax.ShapeDtypeStruct((NN, HH), jnp.float32),
    )(x, W1, dis_col)


def _mid_body(p_ref, g_ref, dis_ref, b_ref, w_ref, h_ref, gn_ref):
    dis = dis_ref[...]
    h = jnp.maximum(dis * (p_ref[0] + p_ref[1] + g_ref[...]) + b_ref[...], 0.0)
    h_ref[...] = h
    gn_ref[...] = dis * jnp.dot(h, w_ref[...],
                                preferred_element_type=jnp.float32)


def _tc_mid(p, g, dis_col, b_row, W_next):
    return pl.pallas_call(
        _mid_body,
        grid=(NN // RB,),
        in_specs=[
            pl.BlockSpec((NC, RB, HH), lambda i: (0, i, 0)),
            pl.BlockSpec((RB, HH), lambda i: (i, 0)),
            pl.BlockSpec((RB, 1), lambda i: (i, 0)),
            pl.BlockSpec((1, HH), lambda i: (0, 0)),
            pl.BlockSpec((HH, HH), lambda i: (0, 0)),
        ],
        out_specs=[
            pl.BlockSpec((RB, HH), lambda i: (i, 0)),
            pl.BlockSpec((RB, HH), lambda i: (i, 0)),
        ],
        out_shape=[
            jax.ShapeDtypeStruct((NN, HH), jnp.float32),
            jax.ShapeDtypeStruct((NN, HH), jnp.float32),
        ],
    )(p, g, dis_col, b_row, W_next)


def _tail_body(p_ref, g_ref, dis_ref, b_ref, h1_ref, h2_ref,
               wa_ref, wb_ref, wc_ref, lb_ref, wcl_ref,
               xl_ref, ghh_ref):
    dis = dis_ref[...]
    h3 = jnp.maximum(dis * (p_ref[0] + p_ref[1] + g_ref[...]) + b_ref[...], 0.0)
    xl = (jnp.dot(h1_ref[...], wa_ref[...], preferred_element_type=jnp.float32)
          + jnp.dot(h2_ref[...], wb_ref[...], preferred_element_type=jnp.float32)
          + jnp.dot(h3, wc_ref[...], preferred_element_type=jnp.float32)
          + lb_ref[...])
    xl = jnp.maximum(xl, 0.0)
    xl_ref[...] = xl
    ghh_ref[...] = dis * jnp.dot(xl, wcl_ref[...],
                                 preferred_element_type=jnp.float32)


def _tc_tail(p, g, dis_col, b_row, h1, h2, wa, wb, wc, lb, wcl):
    return pl.pallas_call(
        _tail_body,
        grid=(NN // RB,),
        in_specs=[
            pl.BlockSpec((NC, RB, HH), lambda i: (0, i, 0)),
            pl.BlockSpec((RB, HH), lambda i: (i, 0)),
            pl.BlockSpec((RB, 1), lambda i: (i, 0)),
            pl.BlockSpec((1, HH), lambda i: (0, 0)),
            pl.BlockSpec((RB, HH), lambda i: (i, 0)),
            pl.BlockSpec((RB, HH), lambda i: (i, 0)),
            pl.BlockSpec((HH, HH), lambda i: (0, 0)),
            pl.BlockSpec((HH, HH), lambda i: (0, 0)),
            pl.BlockSpec((HH, HH), lambda i: (0, 0)),
            pl.BlockSpec((1, HH), lambda i: (0, 0)),
            pl.BlockSpec((HH, 1), lambda i: (0, 0)),
        ],
        out_specs=[
            pl.BlockSpec((RB, HH), lambda i: (i, 0)),
            pl.BlockSpec((RB, 1), lambda i: (i, 0)),
        ],
        out_shape=[
            jax.ShapeDtypeStruct((NN, HH), jnp.float32),
            jax.ShapeDtypeStruct((NN, 1), jnp.float32),
        ],
    )(p, g, dis_col, b_row, h1, h2, wa, wb, wc, lb, wcl)


def _fin_body(q_ref, ghh_ref, dis_ref, bc_ref, xl_ref, w2_ref, b2_ref,
              ch_ref, val_ref):
    nrow = NPAD // 128
    c = dis_ref[...] * (q_ref[0] + q_ref[1] + ghh_ref[...]) + bc_ref[0, 0]
    flat = (lax.broadcasted_iota(jnp.int32, (nrow, 128), 0) * 128
            + lax.broadcasted_iota(jnp.int32, (nrow, 128), 1))
    valid = flat < NN
    c = jnp.where(valid, c, -jnp.inf)
    m = jnp.max(c)
    ex = jnp.exp(c - m)
    ch_ref[...] = ex / jnp.sum(ex)
    v = jnp.mean(xl_ref[...], axis=0, keepdims=True)
    val_ref[...] = jnp.dot(v, w2_ref[...],
                           preferred_element_type=jnp.float32) + b2_ref[...]


def _tc_fin(q, ghh_pad, dis2, bc, xl, lin2_W, lin2_b):
    return pl.pallas_call(
        _fin_body,
        out_shape=[
            jax.ShapeDtypeStruct((NPAD // 128, 128), jnp.float32),
            jax.ShapeDtypeStruct((1, 1), jnp.float32),
        ],
    )(q.reshape(NC, NPAD // 128, 128), ghh_pad.reshape(NPAD // 128, 128),
      dis2, bc.reshape(1, 1), xl, lin2_W, lin2_b.reshape(1, 1))


def kernel(x, edge_index, weight, W1, b1, W2, b2, W3, b3,
           lin1_W, lin1_b, Wc, bc, lin2_W, lin2_b):
    f32 = jnp.float32
    src = edge_index[0]
    dst = edge_index[1]
    w = weight.astype(f32)
    pad = EPAD - EE
    srcp = jnp.concatenate([src, jnp.zeros((pad,), src.dtype)])
    dstp = jnp.concatenate([dst, jnp.zeros((pad,), dst.dtype)])
    wp = jnp.concatenate([w, jnp.zeros((pad,), f32)])
    src2d = srcp.reshape(EPAD // 128, 128)
    dst2d = dstp.reshape(EPAD // 128, 128)
    zeros16 = jnp.zeros((NPAD, HH), f32)

    degp = _sc_deg(dstp, wp)
    dis2 = _tc_dis(degp)
    dis_col = dis2.reshape(NPAD, 1)[:NN]

    g1 = _tc_l1(x, W1, dis_col)
    p1 = _sc_layer(src2d, dst2d, wp, g1, zeros16)
    h1, g2 = _tc_mid(p1, g1, dis_col, b1.reshape(1, HH), W2)
    p2 = _sc_layer(src2d, dst2d, wp, g2, zeros16)
    h2, g3 = _tc_mid(p2, g2, dis_col, b2.reshape(1, HH), W3)
    p3 = _sc_layer(src2d, dst2d, wp, g3, zeros16)

    wa = lin1_W[:HH]
    wb = lin1_W[HH:2 * HH]
    wc = lin1_W[2 * HH:]
    xl, ghh = _tc_tail(p3, g3, dis_col, b3.reshape(1, HH), h1, h2,
                       wa, wb, wc, lin1_b.reshape(1, HH), Wc)

    ghh_pad = jnp.pad(ghh.reshape(NN), (0, NPAD - NN))
    q = _sc_conv1(srcp, dstp, wp, ghh_pad)
    ch2, value = _tc_fin(q, ghh_pad, dis2, bc, xl, lin2_W, lin2_b)
    choice = ch2.reshape(NPAD)[:NN]
    return (choice, value)


# speedup vs baseline: 30.7884x; 1.5484x over previous
import dataclasses
import functools

import jax
import jax.numpy as jnp
from jax import lax
from jax.experimental import pallas as pl
from jax.experimental.pallas import tpu as pltpu
from jax.experimental.pallas import tpu_sc as plsc

NN = 10000
EE = 320000
DD = 128
HH = 16

NC = 2
NS = 16
NW = NC * NS
NPAD = 10240
SLICE = NPAD // NS
EPW = 10240
EPAD = NW * EPW
CH = 1024
NCHK = EPW // CH
RPC = CH // 128
ROWS_PW = EPW // 128

@functools.cache
def _sc_mesh():
    return plsc.VectorSubcoreMesh(core_axis_name="c", subcore_axis_name="s",
                                  num_cores=NC, num_subcores=NS)


def _sc_params():
    cp = pltpu.CompilerParams()
    cp = dataclasses.replace(cp, needs_layout_passes=False,
                             use_tc_tiling_on_sc=False)
    return cp


NBUF = 4


@functools.cache
def _sc_layer_kernel():
    return functools.partial(
        pl.kernel,
        out_type=jax.ShapeDtypeStruct((NC, NPAD, HH), jnp.float32),
        mesh=_sc_mesh(),
        compiler_params=_sc_params(),
        scratch_types=[
            pltpu.VMEM((ROWS_PW, 128), jnp.int32),
            pltpu.VMEM((ROWS_PW, 128), jnp.int32),
            pltpu.VMEM((EPW,), jnp.float32),
            pltpu.VMEM((NBUF, CH, HH), jnp.float32),
            pltpu.VMEM_SHARED((NPAD, HH), jnp.float32),
            pltpu.SemaphoreType.DMA,
            pltpu.SemaphoreType.DMA,
            pltpu.SemaphoreType.DMA,
            pltpu.SemaphoreType.DMA,
        ],
    )(_sc_layer_body)


def _sc_layer(src2d, dst2d, wp, g, z):
    return _sc_layer_kernel()(src2d, dst2d, wp, g, z)


def _sc_layer_body(src_hbm, dst_hbm, w_hbm, g_hbm, z_hbm, out_hbm,
                   si_v, di_v, w_v, rows_v, acc_sh, semz, semi, semg, sems):
    c = lax.axis_index("c")
    s = lax.axis_index("s")
    wid = s * NC + c

    zcp = pltpu.async_copy(z_hbm.at[pl.ds(s * SLICE, SLICE)],
                           acc_sh.at[pl.ds(s * SLICE, SLICE)], semz)
    icps = [
        pltpu.async_copy(src_hbm.at[pl.ds(wid * ROWS_PW, ROWS_PW)], si_v, semi),
        pltpu.async_copy(dst_hbm.at[pl.ds(wid * ROWS_PW, ROWS_PW)], di_v, semi),
        pltpu.async_copy(w_hbm.at[pl.ds(wid * EPW, EPW)], w_v, semi),
    ]
    for cp in icps:
        cp.wait()

    def issue_gathers(t):
        slot = rows_v.at[t % NBUF]
        return [
            pltpu.async_copy(g_hbm.at[si_v.at[t * RPC + j]],
                             slot.at[pl.ds(j * 128, 128)], semg)
            for j in range(RPC)
        ]

    gd = {0: issue_gathers(0), 1: issue_gathers(1)}
    zcp.wait()
    plsc.subcore_barrier()

    sd = {}
    for t in range(NCHK):
        if t >= 2:
            for cp in sd[t - 2]:
                cp.wait()
        if t + 2 < NCHK:
            gd[t + 2] = issue_gathers(t + 2)
        for cp in gd[t]:
            cp.wait()
        slot = t % NBUF
        base_e = t * CH

        @pl.loop(0, CH, step=8)
        def _scale(e0, slot=slot, base_e=base_e):
            for u in range(8):
                e = e0 + u
                spl = plsc.load_gather(
                    w_v, [jnp.full((16,), base_e + e, jnp.int32)])
                rows_v[slot, e, :] = rows_v[slot, e, :] * spl

        sd[t] = [
            pltpu.async_copy(rows_v.at[slot].at[pl.ds(j * 128, 128)],
                             acc_sh.at[di_v.at[t * RPC + j]], sems, add=True)
            for j in range(RPC)
        ]

    for cp in sd[NCHK - 2]:
        cp.wait()
    for cp in sd[NCHK - 1]:
        cp.wait()
    plsc.subcore_barrier()
    pltpu.sync_copy(acc_sh.at[pl.ds(s * SLICE, SLICE)],
                    out_hbm.at[c, pl.ds(s * SLICE, SLICE)])


def _scalar_accumulate_and_reduce(body_per_group, out_hbm,
                                  d_v, w_v, acc_v, red_v, out_v, red_sh,
                                  dst_hbm, w_hbm, load_extra):
    c = lax.axis_index("c")
    s = lax.axis_index("s")
    wid = s * NC + c

    @pl.loop(0, NPAD // 16)
    def _zero(i):
        acc_v[pl.ds(i * 16, 16)] = jnp.zeros((16,), jnp.float32)

    @pl.loop(0, NCHK)
    def _chunk(t):
        e0 = wid * EPW + t * CH
        pltpu.sync_copy(dst_hbm.at[pl.ds(e0, CH)], d_v)
        pltpu.sync_copy(w_hbm.at[pl.ds(e0, CH)], w_v)
        load_extra(e0)

        @pl.loop(0, CH // 16)
        def _grp(i):
            sl = pl.ds(i * 16, 16)
            plsc.addupdate_scatter(acc_v, [d_v[sl]], body_per_group(sl))

    pltpu.sync_copy(acc_v, red_sh.at[s])
    plsc.subcore_barrier()
    for r in range(NS):
        pltpu.sync_copy(red_sh.at[r, pl.ds(s * SLICE, SLICE)], red_v.at[r])

    @pl.loop(0, SLICE // 16)
    def _sum(i):
        sl = pl.ds(i * 16, 16)
        acc16 = red_v[0, sl]
        for r in range(1, NS):
            acc16 = acc16 + red_v[r, sl]
        out_v[sl] = acc16

    pltpu.sync_copy(out_v, out_hbm.at[c, pl.ds(s * SLICE, SLICE)])


_SCALAR_SCRATCH = [
    pltpu.VMEM((CH,), jnp.int32),
    pltpu.VMEM((CH,), jnp.float32),
    pltpu.VMEM((NPAD,), jnp.float32),
    pltpu.VMEM((NS, SLICE), jnp.float32),
    pltpu.VMEM((SLICE,), jnp.float32),
    pltpu.VMEM_SHARED((NS, NPAD), jnp.float32),
    pltpu.SemaphoreType.DMA,
]


def _sc_deg_body(dst_hbm, w_hbm, out_hbm,
                 d_v, w_v, acc_v, red_v, out_v, red_sh, sem):
    _scalar_accumulate_and_reduce(
        lambda sl: w_v[sl], out_hbm,
        d_v, w_v, acc_v, red_v, out_v, red_sh,
        dst_hbm, w_hbm, lambda e0: None)


@functools.cache
def _sc_deg_kernel():
    return functools.partial(
        pl.kernel,
        out_type=jax.ShapeDtypeStruct((NC, NPAD), jnp.float32),
        mesh=_sc_mesh(),
        compiler_params=_sc_params(),
        scratch_types=_SCALAR_SCRATCH,
    )(_sc_deg_body)


def _sc_deg(dstp, wp):
    return _sc_deg_kernel()(dstp, wp)


def _sc_conv1_body(src_hbm, dst_hbm, w_hbm, hh_hbm, out_hbm,
                   src_v, hh_v, d_v, w_v, acc_v, red_v, out_v, red_sh, sem):
    pltpu.sync_copy(hh_hbm, hh_v)

    def load_extra(e0):
        pltpu.sync_copy(src_hbm.at[pl.ds(e0, CH)], src_v)

    _scalar_accumulate_and_reduce(
        lambda sl: w_v[sl] * plsc.load_gather(hh_v, [src_v[sl]]), out_hbm,
        d_v, w_v, acc_v, red_v, out_v, red_sh,
        dst_hbm, w_hbm, load_extra)


@functools.cache
def _sc_conv1_kernel():
    return functools.partial(
        pl.kernel,
        out_type=jax.ShapeDtypeStruct((NC, NPAD), jnp.float32),
        mesh=_sc_mesh(),
        compiler_params=_sc_params(),
        scratch_types=[pltpu.VMEM((CH,), jnp.int32),
                       pltpu.VMEM((NPAD,), jnp.float32)] + _SCALAR_SCRATCH,
    )(_sc_conv1_body)


def _sc_conv1(srcp, dstp, wp, hh):
    return _sc_conv1_kernel()(srcp, dstp, wp, hh)

RB = 1000


def _dis_body(p_ref, o_ref):
    deg = p_ref[0] + p_ref[1] + 1.0
    o_ref[...] = jnp.where(deg > 0, lax.rsqrt(deg), 0.0)


def _tc_dis(degp):
    return pl.pallas_call(
        _dis_body,
        out_shape=jax.ShapeDtypeStruct((NPAD // 128, 128), jnp.float32),
    )(degp.reshape(NC, NPAD // 128, 128))


def _l1_body(x_ref, w_ref, dis_ref, o_ref):
    t = jnp.dot(x_ref[...], w_ref[...], preferred_element_type=jnp.float32)
    o_ref[...] = t * dis_ref[...]


def _tc_l1(x, W1, dis_col):
    return pl.pallas_call(
        _l1_body,
        grid=(NN // RB,),
        in_specs=[
            pl.BlockSpec((RB, DD), lambda i: (i, 0)),
            pl.BlockSpec((DD, HH), lambda i: (0, 0)),
            pl.BlockSpec((RB, 1), lambda i: (i, 0)),
        ],
        out_specs=pl.BlockSpec((RB, HH), lambda i: (i, 0)),
        out_shape=jax.ShapeDtypeStruct((NN, HH), jnp.float32),
    )(x, W1, dis_col)


def _mid_body(p_ref, g_ref, dis_ref, b_ref, w_ref, h_ref, gn_ref):
    dis = dis_ref[...]
    h = jnp.maximum(dis * (p_ref[0] + p_ref[1] + g_ref[...]) + b_ref[...], 0.0)
    h_ref[...] = h
    gn_ref[...] = dis * jnp.dot(h, w_ref[...],
                                preferred_element_type=jnp.float32)


def _tc_mid(p, g, dis_col, b_row, W_next):
    return pl.pallas_call(
        _mid_body,
        grid=(NN // RB,),
        in_specs=[
            pl.BlockSpec((NC, RB, HH), lambda i: (0, i, 0)),
            pl.BlockSpec((RB, HH), lambda i: (i, 0)),
            pl.BlockSpec((RB, 1), lambda i: (i, 0)),
            pl.BlockSpec((1, HH), lambda i: (0, 0)),
            pl.BlockSpec((HH, HH), lambda i: (0, 0)),
        ],
        out_specs=[
            pl.BlockSpec((RB, HH), lambda i: (i, 0)),
            pl.BlockSpec((RB, HH), lambda i: (i, 0)),
        ],
        out_shape=[
            jax.ShapeDtypeStruct((NN, HH), jnp.float32),
            jax.ShapeDtypeStruct((NN, HH), jnp.float32),
        ],
    )(p, g, dis_col, b_row, W_next)


def _tail_body(p_ref, g_ref, dis_ref, b_ref, h1_ref, h2_ref,
               wa_ref, wb_ref, wc_ref, lb_ref, wcl_ref,
               xl_ref, ghh_ref):
    dis = dis_ref[...]
    h3 = jnp.maximum(dis * (p_ref[0] + p_ref[1] + g_ref[...]) + b_ref[...], 0.0)
    xl = (jnp.dot(h1_ref[...], wa_ref[...], preferred_element_type=jnp.float32)
          + jnp.dot(h2_ref[...], wb_ref[...], preferred_element_type=jnp.float32)
          + jnp.dot(h3, wc_ref[...], preferred_element_type=jnp.float32)
          + lb_ref[...])
    xl = jnp.maximum(xl, 0.0)
    xl_ref[...] = xl
    ghh_ref[...] = dis * jnp.dot(xl, wcl_ref[...],
                                 preferred_element_type=jnp.float32)


def _tc_tail(p, g, dis_col, b_row, h1, h2, wa, wb, wc, lb, wcl):
    return pl.pallas_call(
        _tail_body,
        grid=(NN // RB,),
        in_specs=[
            pl.BlockSpec((NC, RB, HH), lambda i: (0, i, 0)),
            pl.BlockSpec((RB, HH), lambda i: (i, 0)),
            pl.BlockSpec((RB, 1), lambda i: (i, 0)),
            pl.BlockSpec((1, HH), lambda i: (0, 0)),
            pl.BlockSpec((RB, HH), lambda i: (i, 0)),
            pl.BlockSpec((RB, HH), lambda i: (i, 0)),
            pl.BlockSpec((HH, HH), lambda i: (0, 0)),
            pl.BlockSpec((HH, HH), lambda i: (0, 0)),
            pl.BlockSpec((HH, HH), lambda i: (0, 0)),
            pl.BlockSpec((1, HH), lambda i: (0, 0)),
            pl.BlockSpec((HH, 1), lambda i: (0, 0)),
        ],
        out_specs=[
            pl.BlockSpec((RB, HH), lambda i: (i, 0)),
            pl.BlockSpec((RB, 1), lambda i: (i, 0)),
        ],
        out_shape=[
            jax.ShapeDtypeStruct((NN, HH), jnp.float32),
            jax.ShapeDtypeStruct((NN, 1), jnp.float32),
        ],
    )(p, g, dis_col, b_row, h1, h2, wa, wb, wc, lb, wcl)


def _fin_body(q_ref, ghh_ref, dis_ref, bc_ref, xl_ref, w2_ref, b2_ref,
              ch_ref, val_ref):
    nrow = NPAD // 128
    c = dis_ref[...] * (q_ref[0] + q_ref[1] + ghh_ref[...]) + bc_ref[0, 0]
    flat = (lax.broadcasted_iota(jnp.int32, (nrow, 128), 0) * 128
            + lax.broadcasted_iota(jnp.int32, (nrow, 128), 1))
    valid = flat < NN
    c = jnp.where(valid, c, -jnp.inf)
    m = jnp.max(c)
    ex = jnp.exp(c - m)
    ch_ref[...] = ex / jnp.sum(ex)
    v = jnp.mean(xl_ref[...], axis=0, keepdims=True)
    val_ref[...] = jnp.dot(v, w2_ref[...],
                           preferred_element_type=jnp.float32) + b2_ref[...]


def _tc_fin(q, ghh_pad, dis2, bc, xl, lin2_W, lin2_b):
    return pl.pallas_call(
        _fin_body,
        out_shape=[
            jax.ShapeDtypeStruct((NPAD // 128, 128), jnp.float32),
            jax.ShapeDtypeStruct((1, 1), jnp.float32),
        ],
    )(q.reshape(NC, NPAD // 128, 128), ghh_pad.reshape(NPAD // 128, 128),
      dis2, bc.reshape(1, 1), xl, lin2_W, lin2_b.reshape(1, 1))


def kernel(x, edge_index, weight, W1, b1, W2, b2, W3, b3,
           lin1_W, lin1_b, Wc, bc, lin2_W, lin2_b):
    f32 = jnp.float32
    src = edge_index[0]
    dst = edge_index[1]
    w = weight.astype(f32)
    pad = EPAD - EE
    srcp = jnp.concatenate([src, jnp.zeros((pad,), src.dtype)])
    dstp = jnp.concatenate([dst, jnp.zeros((pad,), dst.dtype)])
    wp = jnp.concatenate([w, jnp.zeros((pad,), f32)])
    src2d = srcp.reshape(EPAD // 128, 128)
    dst2d = dstp.reshape(EPAD // 128, 128)
    zeros16 = jnp.zeros((NPAD, HH), f32)

    degp = _sc_deg(dstp, wp)
    dis2 = _tc_dis(degp)
    dis_col = dis2.reshape(NPAD, 1)[:NN]

    g1 = _tc_l1(x, W1, dis_col)
    p1 = _sc_layer(src2d, dst2d, wp, g1, zeros16)
    h1, g2 = _tc_mid(p1, g1, dis_col, b1.reshape(1, HH), W2)
    p2 = _sc_layer(src2d, dst2d, wp, g2, zeros16)
    h2, g3 = _tc_mid(p2, g2, dis_col, b2.reshape(1, HH), W3)
    p3 = _sc_layer(src2d, dst2d, wp, g3, zeros16)

    wa = lin1_W[:HH]
    wb = lin1_W[HH:2 * HH]
    wc = lin1_W[2 * HH:]
    xl, ghh = _tc_tail(p3, g3, dis_col, b3.reshape(1, HH), h1, h2,
                       wa, wb, wc, lin1_b.reshape(1, HH), Wc)

    ghh_pad = jnp.pad(ghh.reshape(NN), (0, NPAD - NN))
    q = _sc_conv1(srcp, dstp, wp, ghh_pad)
    ch2, value = _tc_fin(q, ghh_pad, dis2, bc, xl, lin2_W, lin2_b)
    choice = ch2.reshape(NPAD)[:NN]
    return (choice, value)


# speedup vs baseline: 34.6302x; 1.1248x over previous
import dataclasses
import functools

import jax
import jax.numpy as jnp
from jax import lax
from jax.experimental import pallas as pl
from jax.experimental.pallas import tpu as pltpu
from jax.experimental.pallas import tpu_sc as plsc

NN = 10000
EE = 320000
DD = 128
HH = 16

NC = 2
NS = 16
NW = NC * NS
NPAD = 10240
SLICE = NPAD // NS
EPW = 10240
EPAD = NW * EPW
CH = 1024
NCHK = EPW // CH
RPC = CH // 128
ROWS_PW = EPW // 128

@functools.cache
def _sc_mesh():
    return plsc.VectorSubcoreMesh(core_axis_name="c", subcore_axis_name="s",
                                  num_cores=NC, num_subcores=NS)


def _sc_params():
    cp = pltpu.CompilerParams()
    cp = dataclasses.replace(cp, needs_layout_passes=False,
                             use_tc_tiling_on_sc=False)
    return cp


NBUF = 4


def _splat(v16, u):
    idx = jnp.full((16, 1), u, jnp.int32)
    return lax.gather(
        v16, idx,
        lax.GatherDimensionNumbers(offset_dims=(), collapsed_slice_dims=(0,),
                                   start_index_map=(0,)),
        slice_sizes=(1,), mode=lax.GatherScatterMode.PROMISE_IN_BOUNDS)


@functools.cache
def _sc_layer_kernel():
    return functools.partial(
        pl.kernel,
        out_type=jax.ShapeDtypeStruct((NC, NPAD, HH), jnp.float32),
        mesh=_sc_mesh(),
        compiler_params=_sc_params(),
        scratch_types=[
            pltpu.VMEM((ROWS_PW, 128), jnp.int32),
            pltpu.VMEM((ROWS_PW, 128), jnp.int32),
            pltpu.VMEM((EPW,), jnp.float32),
            pltpu.VMEM((NBUF, CH, HH), jnp.float32),
            pltpu.VMEM_SHARED((NPAD, HH), jnp.float32),
            pltpu.SemaphoreType.DMA,
            pltpu.SemaphoreType.DMA,
            pltpu.SemaphoreType.DMA,
            pltpu.SemaphoreType.DMA,
        ],
    )(_sc_layer_body)


def _sc_layer(src2d, dst2d, wp, g, z):
    return _sc_layer_kernel()(src2d, dst2d, wp, g, z)


def _sc_layer_body(src_hbm, dst_hbm, w_hbm, g_hbm, z_hbm, out_hbm,
                   si_v, di_v, w_v, rows_v, acc_sh, semz, semi, semg, sems):
    c = lax.axis_index("c")
    s = lax.axis_index("s")
    wid = s * NC + c

    zcp = pltpu.async_copy(z_hbm.at[pl.ds(s * SLICE, SLICE)],
                           acc_sh.at[pl.ds(s * SLICE, SLICE)], semz)
    icps = [
        pltpu.async_copy(src_hbm.at[pl.ds(wid * ROWS_PW, ROWS_PW)], si_v, semi),
        pltpu.async_copy(dst_hbm.at[pl.ds(wid * ROWS_PW, ROWS_PW)], di_v, semi),
        pltpu.async_copy(w_hbm.at[pl.ds(wid * EPW, EPW)], w_v, semi),
    ]
    for cp in icps:
        cp.wait()

    def issue_gathers(t):
        slot = rows_v.at[t % NBUF]
        return [
            pltpu.async_copy(g_hbm.at[si_v.at[t * RPC + j]],
                             slot.at[pl.ds(j * 128, 128)], semg)
            for j in range(RPC)
        ]

    gd = {0: issue_gathers(0), 1: issue_gathers(1)}
    zcp.wait()
    plsc.subcore_barrier()

    sd = {}
    for t in range(NCHK):
        if t >= 2:
            for cp in sd[t - 2]:
                cp.wait()
        if t + 2 < NCHK:
            gd[t + 2] = issue_gathers(t + 2)
        for cp in gd[t]:
            cp.wait()
        slot = t % NBUF
        base_e = t * CH

        @pl.loop(0, CH, step=16)
        def _scale(e0, slot=slot, base_e=base_e):
            w16 = w_v[pl.ds(base_e + e0, 16)]
            for u in range(16):
                spl = _splat(w16, u)
                e = e0 + u
                rows_v[slot, e, :] = rows_v[slot, e, :] * spl

        sd[t] = [
            pltpu.async_copy(rows_v.at[slot].at[pl.ds(j * 128, 128)],
                             acc_sh.at[di_v.at[t * RPC + j]], sems, add=True)
            for j in range(RPC)
        ]

    for cp in sd[NCHK - 2]:
        cp.wait()
    for cp in sd[NCHK - 1]:
        cp.wait()
    plsc.subcore_barrier()
    pltpu.sync_copy(acc_sh.at[pl.ds(s * SLICE, SLICE)],
                    out_hbm.at[c, pl.ds(s * SLICE, SLICE)])


def _scalar_accumulate_and_reduce(body_per_group, out_hbm,
                                  d_v, w_v, acc_v, red_v, out_v, red_sh, sem,
                                  extra_cps):
    c = lax.axis_index("c")
    s = lax.axis_index("s")
    wid = s * NC + c

    @pl.loop(0, NPAD, step=128)
    def _zero(i):
        for u in range(8):
            acc_v[pl.ds(i + u * 16, 16)] = jnp.zeros((16,), jnp.float32)

    for cp in extra_cps:
        cp.wait()

    @pl.loop(0, EPW, step=64)
    def _grp(e0):
        for u in range(4):
            sl = pl.ds(e0 + u * 16, 16)
            plsc.addupdate_scatter(acc_v, [d_v[sl]], body_per_group(sl))

    pltpu.sync_copy(acc_v, red_sh.at[s])
    plsc.subcore_barrier()
    rcps = [
        pltpu.async_copy(red_sh.at[r, pl.ds(s * SLICE, SLICE)],
                         red_v.at[r], sem)
        for r in range(NS)
    ]
    for cp in rcps:
        cp.wait()

    @pl.loop(0, SLICE, step=16)
    def _sum(i):
        sl = pl.ds(i, 16)
        acc16 = red_v[0, sl]
        for r in range(1, NS):
            acc16 = acc16 + red_v[r, sl]
        out_v[sl] = acc16

    pltpu.sync_copy(out_v, out_hbm.at[c, pl.ds(s * SLICE, SLICE)])


_SCALAR_SCRATCH = [
    pltpu.VMEM((EPW,), jnp.int32),
    pltpu.VMEM((EPW,), jnp.float32),
    pltpu.VMEM((NPAD,), jnp.float32),
    pltpu.VMEM((NS, SLICE), jnp.float32),
    pltpu.VMEM((SLICE,), jnp.float32),
    pltpu.VMEM_SHARED((NS, NPAD), jnp.float32),
    pltpu.SemaphoreType.DMA,
]


def _sc_deg_body(dst_hbm, w_hbm, out_hbm,
                 d_v, w_v, acc_v, red_v, out_v, red_sh, sem):
    c = lax.axis_index("c")
    s = lax.axis_index("s")
    wid = s * NC + c
    cps = [
        pltpu.async_copy(dst_hbm.at[pl.ds(wid * EPW, EPW)], d_v, sem),
        pltpu.async_copy(w_hbm.at[pl.ds(wid * EPW, EPW)], w_v, sem),
    ]
    _scalar_accumulate_and_reduce(
        lambda sl: w_v[sl], out_hbm,
        d_v, w_v, acc_v, red_v, out_v, red_sh, sem, cps)


@functools.cache
def _sc_deg_kernel():
    return functools.partial(
        pl.kernel,
        out_type=jax.ShapeDtypeStruct((NC, NPAD), jnp.float32),
        mesh=_sc_mesh(),
        compiler_params=_sc_params(),
        scratch_types=_SCALAR_SCRATCH,
    )(_sc_deg_body)


def _sc_deg(dstp, wp):
    return _sc_deg_kernel()(dstp, wp)


def _sc_conv1_body(src_hbm, dst_hbm, w_hbm, hh_hbm, out_hbm,
                   src_v, hh_v, d_v, w_v, acc_v, red_v, out_v, red_sh, sem):
    c = lax.axis_index("c")
    s = lax.axis_index("s")
    wid = s * NC + c
    cps = [
        pltpu.async_copy(src_hbm.at[pl.ds(wid * EPW, EPW)], src_v, sem),
        pltpu.async_copy(dst_hbm.at[pl.ds(wid * EPW, EPW)], d_v, sem),
        pltpu.async_copy(w_hbm.at[pl.ds(wid * EPW, EPW)], w_v, sem),
        pltpu.async_copy(hh_hbm, hh_v, sem),
    ]
    _scalar_accumulate_and_reduce(
        lambda sl: w_v[sl] * plsc.load_gather(hh_v, [src_v[sl]]), out_hbm,
        d_v, w_v, acc_v, red_v, out_v, red_sh, sem, cps)


@functools.cache
def _sc_conv1_kernel():
    return functools.partial(
        pl.kernel,
        out_type=jax.ShapeDtypeStruct((NC, NPAD), jnp.float32),
        mesh=_sc_mesh(),
        compiler_params=_sc_params(),
        scratch_types=[pltpu.VMEM((EPW,), jnp.int32),
                       pltpu.VMEM((NPAD,), jnp.float32)] + _SCALAR_SCRATCH,
    )(_sc_conv1_body)


def _sc_conv1(srcp, dstp, wp, hh):
    return _sc_conv1_kernel()(srcp, dstp, wp, hh)

RB = 1000


def _dis_body(p_ref, o_ref):
    deg = p_ref[0] + p_ref[1] + 1.0
    o_ref[...] = jnp.where(deg > 0, lax.rsqrt(deg), 0.0)


def _tc_dis(degp):
    return pl.pallas_call(
        _dis_body,
        out_shape=jax.ShapeDtypeStruct((NPAD // 128, 128), jnp.float32),
    )(degp.reshape(NC, NPAD // 128, 128))


def _l1_body(x_ref, w_ref, dis_ref, o_ref):
    t = jnp.dot(x_ref[...], w_ref[...], preferred_element_type=jnp.float32)
    o_ref[...] = t * dis_ref[...]


def _tc_l1(x, W1, dis_col):
    return pl.pallas_call(
        _l1_body,
        grid=(NN // RB,),
        in_specs=[
            pl.BlockSpec((RB, DD), lambda i: (i, 0)),
            pl.BlockSpec((DD, HH), lambda i: (0, 0)),
            pl.BlockSpec((RB, 1), lambda i: (i, 0)),
        ],
        out_specs=pl.BlockSpec((RB, HH), lambda i: (i, 0)),
        out_shape=jax.ShapeDtypeStruct((NN, HH), jnp.float32),
    )(x, W1, dis_col)


def _mid_body(p_ref, g_ref, dis_ref, b_ref, w_ref, h_ref, gn_ref):
    dis = dis_ref[...]
    h = jnp.maximum(dis * (p_ref[0] + p_ref[1] + g_ref[...]) + b_ref[...], 0.0)
    h_ref[...] = h
    gn_ref[...] = dis * jnp.dot(h, w_ref[...],
                                preferred_element_type=jnp.float32)


def _tc_mid(p, g, dis_col, b_row, W_next):
    return pl.pallas_call(
        _mid_body,
        grid=(NN // RB,),
        in_specs=[
            pl.BlockSpec((NC, RB, HH), lambda i: (0, i, 0)),
            pl.BlockSpec((RB, HH), lambda i: (i, 0)),
            pl.BlockSpec((RB, 1), lambda i: (i, 0)),
            pl.BlockSpec((1, HH), lambda i: (0, 0)),
            pl.BlockSpec((HH, HH), lambda i: (0, 0)),
        ],
        out_specs=[
            pl.BlockSpec((RB, HH), lambda i: (i, 0)),
            pl.BlockSpec((RB, HH), lambda i: (i, 0)),
        ],
        out_shape=[
            jax.ShapeDtypeStruct((NN, HH), jnp.float32),
            jax.ShapeDtypeStruct((NN, HH), jnp.float32),
        ],
    )(p, g, dis_col, b_row, W_next)


def _tail_body(p_ref, g_ref, dis_ref, b_ref, h1_ref, h2_ref,
               wa_ref, wb_ref, wc_ref, lb_ref, wcl_ref,
               xl_ref, ghh_ref):
    dis = dis_ref[...]
    h3 = jnp.maximum(dis * (p_ref[0] + p_ref[1] + g_ref[...]) + b_ref[...], 0.0)
    xl = (jnp.dot(h1_ref[...], wa_ref[...], preferred_element_type=jnp.float32)
          + jnp.dot(h2_ref[...], wb_ref[...], preferred_element_type=jnp.float32)
          + jnp.dot(h3, wc_ref[...], preferred_element_type=jnp.float32)
          + lb_ref[...])
    xl = jnp.maximum(xl, 0.0)
    xl_ref[...] = xl
    ghh_ref[...] = dis * jnp.dot(xl, wcl_ref[...],
                                 preferred_element_type=jnp.float32)


def _tc_tail(p, g, dis_col, b_row, h1, h2, wa, wb, wc, lb, wcl):
    return pl.pallas_call(
        _tail_body,
        grid=(NN // RB,),
        in_specs=[
            pl.BlockSpec((NC, RB, HH), lambda i: (0, i, 0)),
            pl.BlockSpec((RB, HH), lambda i: (i, 0)),
            pl.BlockSpec((RB, 1), lambda i: (i, 0)),
            pl.BlockSpec((1, HH), lambda i: (0, 0)),
            pl.BlockSpec((RB, HH), lambda i: (i, 0)),
            pl.BlockSpec((RB, HH), lambda i: (i, 0)),
            pl.BlockSpec((HH, HH), lambda i: (0, 0)),
            pl.BlockSpec((HH, HH), lambda i: (0, 0)),
            pl.BlockSpec((HH, HH), lambda i: (0, 0)),
            pl.BlockSpec((1, HH), lambda i: (0, 0)),
            pl.BlockSpec((HH, 1), lambda i: (0, 0)),
        ],
        out_specs=[
            pl.BlockSpec((RB, HH), lambda i: (i, 0)),
            pl.BlockSpec((RB, 1), lambda i: (i, 0)),
        ],
        out_shape=[
            jax.ShapeDtypeStruct((NN, HH), jnp.float32),
            jax.ShapeDtypeStruct((NN, 1), jnp.float32),
        ],
    )(p, g, dis_col, b_row, h1, h2, wa, wb, wc, lb, wcl)


def _fin_body(q_ref, ghh_ref, dis_ref, bc_ref, xl_ref, w2_ref, b2_ref,
              ch_ref, val_ref):
    nrow = NPAD // 128
    c = dis_ref[...] * (q_ref[0] + q_ref[1] + ghh_ref[...]) + bc_ref[0, 0]
    flat = (lax.broadcasted_iota(jnp.int32, (nrow, 128), 0) * 128
            + lax.broadcasted_iota(jnp.int32, (nrow, 128), 1))
    valid = flat < NN
    c = jnp.where(valid, c, -jnp.inf)
    m = jnp.max(c)
    ex = jnp.exp(c - m)
    ch_ref[...] = ex / jnp.sum(ex)
    v = jnp.mean(xl_ref[...], axis=0, keepdims=True)
    val_ref[...] = jnp.dot(v, w2_ref[...],
                           preferred_element_type=jnp.float32) + b2_ref[...]


def _tc_fin(q, ghh_pad, dis2, bc, xl, lin2_W, lin2_b):
    return pl.pallas_call(
        _fin_body,
        out_shape=[
            jax.ShapeDtypeStruct((NPAD // 128, 128), jnp.float32),
            jax.ShapeDtypeStruct((1, 1), jnp.float32),
        ],
    )(q.reshape(NC, NPAD // 128, 128), ghh_pad.reshape(NPAD // 128, 128),
      dis2, bc.reshape(1, 1), xl, lin2_W, lin2_b.reshape(1, 1))


def kernel(x, edge_index, weight, W1, b1, W2, b2, W3, b3,
           lin1_W, lin1_b, Wc, bc, lin2_W, lin2_b):
    f32 = jnp.float32
    src = edge_index[0]
    dst = edge_index[1]
    w = weight.astype(f32)
    pad = EPAD - EE
    srcp = jnp.concatenate([src, jnp.zeros((pad,), src.dtype)])
    dstp = jnp.concatenate([dst, jnp.zeros((pad,), dst.dtype)])
    wp = jnp.concatenate([w, jnp.zeros((pad,), f32)])
    src2d = srcp.reshape(EPAD // 128, 128)
    dst2d = dstp.reshape(EPAD // 128, 128)
    zeros16 = jnp.zeros((NPAD, HH), f32)

    degp = _sc_deg(dstp, wp)
    dis2 = _tc_dis(degp)
    dis_col = dis2.reshape(NPAD, 1)[:NN]

    g1 = _tc_l1(x, W1, dis_col)
    p1 = _sc_layer(src2d, dst2d, wp, g1, zeros16)
    h1, g2 = _tc_mid(p1, g1, dis_col, b1.reshape(1, HH), W2)
    p2 = _sc_layer(src2d, dst2d, wp, g2, zeros16)
    h2, g3 = _tc_mid(p2, g2, dis_col, b2.reshape(1, HH), W3)
    p3 = _sc_layer(src2d, dst2d, wp, g3, zeros16)

    wa = lin1_W[:HH]
    wb = lin1_W[HH:2 * HH]
    wc = lin1_W[2 * HH:]
    xl, ghh = _tc_tail(p3, g3, dis_col, b3.reshape(1, HH), h1, h2,
                       wa, wb, wc, lin1_b.reshape(1, HH), Wc)

    ghh_pad = jnp.pad(ghh.reshape(NN), (0, NPAD - NN))
    q = _sc_conv1(srcp, dstp, wp, ghh_pad)
    ch2, value = _tc_fin(q, ghh_pad, dis2, bc, xl, lin2_W, lin2_b)
    choice = ch2.reshape(NPAD)[:NN]
    return (choice, value)


# speedup vs baseline: 49.9298x; 1.4418x over previous
import dataclasses
import functools

import jax
import jax.numpy as jnp
from jax import lax
from jax.experimental import pallas as pl
from jax.experimental.pallas import tpu as pltpu
from jax.experimental.pallas import tpu_sc as plsc

NN = 10000
EE = 320000
DD = 128
HH = 16

NC = 2
NS = 16
NW = NC * NS
NPAD = 10240
SLICE = NPAD // NS
EPW = 10240
EPAD = NW * EPW
CH = 1024
NCHK = EPW // CH
RPC = CH // 128
ROWS_PW = EPW // 128

@functools.cache
def _sc_mesh():
    return plsc.VectorSubcoreMesh(core_axis_name="c", subcore_axis_name="s",
                                  num_cores=NC, num_subcores=NS)


def _sc_params():
    cp = pltpu.CompilerParams()
    cp = dataclasses.replace(cp, needs_layout_passes=False,
                             use_tc_tiling_on_sc=False)
    return cp


NBUF = 4


def _splat(v16, u):
    idx = jnp.full((16, 1), u, jnp.int32)
    return lax.gather(
        v16, idx,
        lax.GatherDimensionNumbers(offset_dims=(), collapsed_slice_dims=(0,),
                                   start_index_map=(0,)),
        slice_sizes=(1,), mode=lax.GatherScatterMode.PROMISE_IN_BOUNDS)


@functools.cache
def _sc_layer_kernel():
    return functools.partial(
        pl.kernel,
        out_type=jax.ShapeDtypeStruct((NC, NPAD, HH), jnp.float32),
        mesh=_sc_mesh(),
        compiler_params=_sc_params(),
        scratch_types=[
            pltpu.VMEM((ROWS_PW, 128), jnp.int32),
            pltpu.VMEM((ROWS_PW, 128), jnp.int32),
            pltpu.VMEM((EPW,), jnp.float32),
            pltpu.VMEM((NBUF, CH, HH), jnp.float32),
            pltpu.VMEM_SHARED((NPAD, HH), jnp.float32),
            pltpu.VMEM_SHARED((NN, HH), jnp.float32),
            pltpu.SemaphoreType.DMA,
            pltpu.SemaphoreType.DMA,
            pltpu.SemaphoreType.DMA,
            pltpu.SemaphoreType.DMA,
        ],
    )(_sc_layer_body)


def _sc_layer(src2d, dst2d, wp, g, z):
    return _sc_layer_kernel()(src2d, dst2d, wp, g, z)


GROWS = NN // NS


def _sc_layer_body(src_hbm, dst_hbm, w_hbm, g_hbm, z_hbm, out_hbm,
                   si_v, di_v, w_v, rows_v, acc_sh, g_sh,
                   semz, semi, semg, sems):
    c = lax.axis_index("c")
    s = lax.axis_index("s")
    wid = s * NC + c

    zcp = pltpu.async_copy(z_hbm.at[pl.ds(s * SLICE, SLICE)],
                           acc_sh.at[pl.ds(s * SLICE, SLICE)], semz)
    gcp = pltpu.async_copy(g_hbm.at[pl.ds(s * GROWS, GROWS)],
                           g_sh.at[pl.ds(s * GROWS, GROWS)], semz)
    icps = [
        pltpu.async_copy(src_hbm.at[pl.ds(wid * ROWS_PW, ROWS_PW)], si_v, semi),
        pltpu.async_copy(dst_hbm.at[pl.ds(wid * ROWS_PW, ROWS_PW)], di_v, semi),
        pltpu.async_copy(w_hbm.at[pl.ds(wid * EPW, EPW)], w_v, semi),
    ]
    for cp in icps:
        cp.wait()

    def issue_gathers(t):
        slot = rows_v.at[t % NBUF]
        return [
            pltpu.async_copy(g_sh.at[si_v.at[t * RPC + j]],
                             slot.at[pl.ds(j * 128, 128)], semg)
            for j in range(RPC)
        ]

    zcp.wait()
    gcp.wait()
    plsc.subcore_barrier()
    gd = {0: issue_gathers(0), 1: issue_gathers(1)}

    sd = {}
    for t in range(NCHK):
        if t >= 2:
            for cp in sd[t - 2]:
                cp.wait()
        if t + 2 < NCHK:
            gd[t + 2] = issue_gathers(t + 2)
        for cp in gd[t]:
            cp.wait()
        slot = t % NBUF
        base_e = t * CH

        @pl.loop(0, CH, step=16)
        def _scale(e0, slot=slot, base_e=base_e):
            w16 = w_v[pl.ds(base_e + e0, 16)]
            for u in range(16):
                spl = _splat(w16, u)
                e = e0 + u
                rows_v[slot, e, :] = rows_v[slot, e, :] * spl

        sd[t] = [
            pltpu.async_copy(rows_v.at[slot].at[pl.ds(j * 128, 128)],
                             acc_sh.at[di_v.at[t * RPC + j]], sems, add=True)
            for j in range(RPC)
        ]

    for cp in sd[NCHK - 2]:
        cp.wait()
    for cp in sd[NCHK - 1]:
        cp.wait()
    plsc.subcore_barrier()
    pltpu.sync_copy(acc_sh.at[pl.ds(s * SLICE, SLICE)],
                    out_hbm.at[c, pl.ds(s * SLICE, SLICE)])


def _scalar_accumulate_and_reduce(body_per_group, out_hbm,
                                  d_v, w_v, acc_v, red_v, out_v, red_sh, sem,
                                  extra_cps):
    c = lax.axis_index("c")
    s = lax.axis_index("s")
    wid = s * NC + c

    @pl.loop(0, NPAD, step=128)
    def _zero(i):
        for u in range(8):
            acc_v[pl.ds(i + u * 16, 16)] = jnp.zeros((16,), jnp.float32)

    for cp in extra_cps:
        cp.wait()

    @pl.loop(0, EPW, step=64)
    def _grp(e0):
        for u in range(4):
            sl = pl.ds(e0 + u * 16, 16)
            plsc.addupdate_scatter(acc_v, [d_v[sl]], body_per_group(sl))

    pltpu.sync_copy(acc_v, red_sh.at[s])
    plsc.subcore_barrier()
    rcps = [
        pltpu.async_copy(red_sh.at[r, pl.ds(s * SLICE, SLICE)],
                         red_v.at[r], sem)
        for r in range(NS)
    ]
    for cp in rcps:
        cp.wait()

    @pl.loop(0, SLICE, step=16)
    def _sum(i):
        sl = pl.ds(i, 16)
        acc16 = red_v[0, sl]
        for r in range(1, NS):
            acc16 = acc16 + red_v[r, sl]
        out_v[sl] = acc16

    pltpu.sync_copy(out_v, out_hbm.at[c, pl.ds(s * SLICE, SLICE)])


_SCALAR_SCRATCH = [
    pltpu.VMEM((EPW,), jnp.int32),
    pltpu.VMEM((EPW,), jnp.float32),
    pltpu.VMEM((NPAD,), jnp.float32),
    pltpu.VMEM((NS, SLICE), jnp.float32),
    pltpu.VMEM((SLICE,), jnp.float32),
    pltpu.VMEM_SHARED((NS, NPAD), jnp.float32),
    pltpu.SemaphoreType.DMA,
]


def _sc_deg_body(dst_hbm, w_hbm, out_hbm,
                 d_v, w_v, acc_v, red_v, out_v, red_sh, sem):
    c = lax.axis_index("c")
    s = lax.axis_index("s")
    wid = s * NC + c
    cps = [
        pltpu.async_copy(dst_hbm.at[pl.ds(wid * EPW, EPW)], d_v, sem),
        pltpu.async_copy(w_hbm.at[pl.ds(wid * EPW, EPW)], w_v, sem),
    ]
    _scalar_accumulate_and_reduce(
        lambda sl: w_v[sl], out_hbm,
        d_v, w_v, acc_v, red_v, out_v, red_sh, sem, cps)


@functools.cache
def _sc_deg_kernel():
    return functools.partial(
        pl.kernel,
        out_type=jax.ShapeDtypeStruct((NC, NPAD), jnp.float32),
        mesh=_sc_mesh(),
        compiler_params=_sc_params(),
        scratch_types=_SCALAR_SCRATCH,
    )(_sc_deg_body)


def _sc_deg(dstp, wp):
    return _sc_deg_kernel()(dstp, wp)


def _sc_conv1_body(src_hbm, dst_hbm, w_hbm, hh_hbm, out_hbm,
                   src_v, hh_v, d_v, w_v, acc_v, red_v, out_v, red_sh, sem):
    c = lax.axis_index("c")
    s = lax.axis_index("s")
    wid = s * NC + c
    cps = [
        pltpu.async_copy(src_hbm.at[pl.ds(wid * EPW, EPW)], src_v, sem),
        pltpu.async_copy(dst_hbm.at[pl.ds(wid * EPW, EPW)], d_v, sem),
        pltpu.async_copy(w_hbm.at[pl.ds(wid * EPW, EPW)], w_v, sem),
        pltpu.async_copy(hh_hbm, hh_v, sem),
    ]
    _scalar_accumulate_and_reduce(
        lambda sl: w_v[sl] * plsc.load_gather(hh_v, [src_v[sl]]), out_hbm,
        d_v, w_v, acc_v, red_v, out_v, red_sh, sem, cps)


@functools.cache
def _sc_conv1_kernel():
    return functools.partial(
        pl.kernel,
        out_type=jax.ShapeDtypeStruct((NC, NPAD), jnp.float32),
        mesh=_sc_mesh(),
        compiler_params=_sc_params(),
        scratch_types=[pltpu.VMEM((EPW,), jnp.int32),
                       pltpu.VMEM((NPAD,), jnp.float32)] + _SCALAR_SCRATCH,
    )(_sc_conv1_body)


def _sc_conv1(srcp, dstp, wp, hh):
    return _sc_conv1_kernel()(srcp, dstp, wp, hh)

RB = 1000


def _t1_body(x_ref, w_ref, o_ref):
    o_ref[...] = jnp.dot(x_ref[...], w_ref[...],
                         preferred_element_type=jnp.float32)


def _tc_t1(x, W1):
    return pl.pallas_call(
        _t1_body,
        grid=(NN // RB,),
        in_specs=[
            pl.BlockSpec((RB, DD), lambda i: (i, 0)),
            pl.BlockSpec((DD, HH), lambda i: (0, 0)),
        ],
        out_specs=pl.BlockSpec((RB, HH), lambda i: (i, 0)),
        out_shape=jax.ShapeDtypeStruct((NN, HH), jnp.float32),
    )(x, W1)


def _disg1_body(p_ref, t_ref, g_ref, dis_ref):
    deg = p_ref[0] + p_ref[1] + 1.0
    dis = jnp.where(deg > 0, lax.rsqrt(deg), 0.0)
    dis_ref[...] = dis
    g_ref[...] = t_ref[...] * dis


def _tc_disg1(degp3, t1):
    return pl.pallas_call(
        _disg1_body,
        grid=(NN // RB,),
        in_specs=[
            pl.BlockSpec((NC, RB, 1), lambda i: (0, i, 0)),
            pl.BlockSpec((RB, HH), lambda i: (i, 0)),
        ],
        out_specs=[
            pl.BlockSpec((RB, HH), lambda i: (i, 0)),
            pl.BlockSpec((RB, 1), lambda i: (i, 0)),
        ],
        out_shape=[
            jax.ShapeDtypeStruct((NN, HH), jnp.float32),
            jax.ShapeDtypeStruct((NN, 1), jnp.float32),
        ],
    )(degp3, t1)


def _mid_body(p_ref, g_ref, dis_ref, b_ref, w_ref, h_ref, gn_ref):
    dis = dis_ref[...]
    h = jnp.maximum(dis * (p_ref[0] + p_ref[1] + g_ref[...]) + b_ref[...], 0.0)
    h_ref[...] = h
    gn_ref[...] = dis * jnp.dot(h, w_ref[...],
                                preferred_element_type=jnp.float32)


def _tc_mid(p, g, dis_col, b_row, W_next):
    return pl.pallas_call(
        _mid_body,
        grid=(NN // RB,),
        in_specs=[
            pl.BlockSpec((NC, RB, HH), lambda i: (0, i, 0)),
            pl.BlockSpec((RB, HH), lambda i: (i, 0)),
            pl.BlockSpec((RB, 1), lambda i: (i, 0)),
            pl.BlockSpec((1, HH), lambda i: (0, 0)),
            pl.BlockSpec((HH, HH), lambda i: (0, 0)),
        ],
        out_specs=[
            pl.BlockSpec((RB, HH), lambda i: (i, 0)),
            pl.BlockSpec((RB, HH), lambda i: (i, 0)),
        ],
        out_shape=[
            jax.ShapeDtypeStruct((NN, HH), jnp.float32),
            jax.ShapeDtypeStruct((NN, HH), jnp.float32),
        ],
    )(p, g, dis_col, b_row, W_next)


def _tail_body(p_ref, g_ref, dis_ref, b_ref, h1_ref, h2_ref,
               wa_ref, wb_ref, wc_ref, lb_ref, wcl_ref,
               xl_ref, ghh_ref):
    dis = dis_ref[...]
    h3 = jnp.maximum(dis * (p_ref[0] + p_ref[1] + g_ref[...]) + b_ref[...], 0.0)
    xl = (jnp.dot(h1_ref[...], wa_ref[...], preferred_element_type=jnp.float32)
          + jnp.dot(h2_ref[...], wb_ref[...], preferred_element_type=jnp.float32)
          + jnp.dot(h3, wc_ref[...], preferred_element_type=jnp.float32)
          + lb_ref[...])
    xl = jnp.maximum(xl, 0.0)
    xl_ref[...] = xl
    ghh_ref[...] = dis * jnp.dot(xl, wcl_ref[...],
                                 preferred_element_type=jnp.float32)


def _tc_tail(p, g, dis_col, b_row, h1, h2, wa, wb, wc, lb, wcl):
    return pl.pallas_call(
        _tail_body,
        grid=(NN // RB,),
        in_specs=[
            pl.BlockSpec((NC, RB, HH), lambda i: (0, i, 0)),
            pl.BlockSpec((RB, HH), lambda i: (i, 0)),
            pl.BlockSpec((RB, 1), lambda i: (i, 0)),
            pl.BlockSpec((1, HH), lambda i: (0, 0)),
            pl.BlockSpec((RB, HH), lambda i: (i, 0)),
            pl.BlockSpec((RB, HH), lambda i: (i, 0)),
            pl.BlockSpec((HH, HH), lambda i: (0, 0)),
            pl.BlockSpec((HH, HH), lambda i: (0, 0)),
            pl.BlockSpec((HH, HH), lambda i: (0, 0)),
            pl.BlockSpec((1, HH), lambda i: (0, 0)),
            pl.BlockSpec((HH, 1), lambda i: (0, 0)),
        ],
        out_specs=[
            pl.BlockSpec((RB, HH), lambda i: (i, 0)),
            pl.BlockSpec((RB, 1), lambda i: (i, 0)),
        ],
        out_shape=[
            jax.ShapeDtypeStruct((NN, HH), jnp.float32),
            jax.ShapeDtypeStruct((NN, 1), jnp.float32),
        ],
    )(p, g, dis_col, b_row, h1, h2, wa, wb, wc, lb, wcl)


def _fin_body(q_ref, ghh_ref, dis_ref, bc_ref, xl_ref, w2_ref, b2_ref,
              ch_ref, val_ref):
    nrow = NPAD // 128
    c = dis_ref[...] * (q_ref[0] + q_ref[1] + ghh_ref[...]) + bc_ref[0, 0]
    flat = (lax.broadcasted_iota(jnp.int32, (nrow, 128), 0) * 128
            + lax.broadcasted_iota(jnp.int32, (nrow, 128), 1))
    valid = flat < NN
    c = jnp.where(valid, c, -jnp.inf)
    m = jnp.max(c)
    ex = jnp.exp(c - m)
    ch_ref[...] = ex / jnp.sum(ex)
    v = jnp.mean(xl_ref[...], axis=0, keepdims=True)
    val_ref[...] = jnp.dot(v, w2_ref[...],
                           preferred_element_type=jnp.float32) + b2_ref[...]


def _tc_fin(q, ghh_pad, dis2, bc, xl, lin2_W, lin2_b):
    return pl.pallas_call(
        _fin_body,
        out_shape=[
            jax.ShapeDtypeStruct((NPAD // 128, 128), jnp.float32),
            jax.ShapeDtypeStruct((1, 1), jnp.float32),
        ],
    )(q.reshape(NC, NPAD // 128, 128), ghh_pad.reshape(NPAD // 128, 128),
      dis2, bc.reshape(1, 1), xl, lin2_W, lin2_b.reshape(1, 1))


def kernel(x, edge_index, weight, W1, b1, W2, b2, W3, b3,
           lin1_W, lin1_b, Wc, bc, lin2_W, lin2_b):
    f32 = jnp.float32
    src = edge_index[0]
    dst = edge_index[1]
    w = weight.astype(f32)
    pad = EPAD - EE
    srcp = jnp.concatenate([src, jnp.zeros((pad,), src.dtype)])
    dstp = jnp.concatenate([dst, jnp.zeros((pad,), dst.dtype)])
    wp = jnp.concatenate([w, jnp.zeros((pad,), f32)])
    src2d = srcp.reshape(EPAD // 128, 128)
    dst2d = dstp.reshape(EPAD // 128, 128)
    zeros16 = jnp.zeros((NPAD, HH), f32)

    t1 = _tc_t1(x, W1)
    degp = _sc_deg(dstp, wp)
    g1, dis_col = _tc_disg1(degp.reshape(NC, NPAD, 1), t1)
    dis2 = jnp.pad(dis_col, ((0, NPAD - NN), (0, 0))).reshape(NPAD // 128, 128)
    p1 = _sc_layer(src2d, dst2d, wp, g1, zeros16)
    h1, g2 = _tc_mid(p1, g1, dis_col, b1.reshape(1, HH), W2)
    p2 = _sc_layer(src2d, dst2d, wp, g2, zeros16)
    h2, g3 = _tc_mid(p2, g2, dis_col, b2.reshape(1, HH), W3)
    p3 = _sc_layer(src2d, dst2d, wp, g3, zeros16)

    wa = lin1_W[:HH]
    wb = lin1_W[HH:2 * HH]
    wc = lin1_W[2 * HH:]
    xl, ghh = _tc_tail(p3, g3, dis_col, b3.reshape(1, HH), h1, h2,
                       wa, wb, wc, lin1_b.reshape(1, HH), Wc)

    ghh_pad = jnp.pad(ghh.reshape(NN), (0, NPAD - NN))
    q = _sc_conv1(srcp, dstp, wp, ghh_pad)
    ch2, value = _tc_fin(q, ghh_pad, dis2, bc, xl, lin2_W, lin2_b)
    choice = ch2.reshape(NPAD)[:NN]
    return (choice, value)


# speedup vs baseline: 51.2118x; 1.0257x over previous
import dataclasses
import functools

import jax
import jax.numpy as jnp
from jax import lax
from jax.experimental import pallas as pl
from jax.experimental.pallas import tpu as pltpu
from jax.experimental.pallas import tpu_sc as plsc

NN = 10000
EE = 320000
DD = 128
HH = 16

NC = 2
NS = 16
NW = NC * NS
NPAD = 10240
SLICE = NPAD // NS
EPW = 10240
EPAD = NW * EPW
CH = 1024
NCHK = EPW // CH
RPC = CH // 128
ROWS_PW = EPW // 128

@functools.cache
def _sc_mesh():
    return plsc.VectorSubcoreMesh(core_axis_name="c", subcore_axis_name="s",
                                  num_cores=NC, num_subcores=NS)


def _sc_params():
    cp = pltpu.CompilerParams()
    cp = dataclasses.replace(cp, needs_layout_passes=False,
                             use_tc_tiling_on_sc=False)
    return cp


NBUF = 4


def _splat(v16, u):
    idx = jnp.full((16, 1), u, jnp.int32)
    return lax.gather(
        v16, idx,
        lax.GatherDimensionNumbers(offset_dims=(), collapsed_slice_dims=(0,),
                                   start_index_map=(0,)),
        slice_sizes=(1,), mode=lax.GatherScatterMode.PROMISE_IN_BOUNDS)


@functools.cache
def _sc_layer_kernel():
    return functools.partial(
        pl.kernel,
        out_type=jax.ShapeDtypeStruct((NC, NPAD, HH), jnp.float32),
        mesh=_sc_mesh(),
        compiler_params=_sc_params(),
        scratch_types=[
            pltpu.VMEM((ROWS_PW, 128), jnp.int32),
            pltpu.VMEM((ROWS_PW, 128), jnp.int32),
            pltpu.VMEM((EPW,), jnp.float32),
            pltpu.VMEM((NBUF, CH, HH), jnp.float32),
            pltpu.VMEM_SHARED((NPAD, HH), jnp.float32),
            pltpu.VMEM_SHARED((NN, HH), jnp.float32),
            pltpu.SemaphoreType.DMA,
            pltpu.SemaphoreType.DMA,
            pltpu.SemaphoreType.DMA,
            pltpu.SemaphoreType.DMA,
        ],
    )(_sc_layer_body)


def _sc_layer(src2d, dst2d, wp, g, z):
    return _sc_layer_kernel()(src2d, dst2d, wp, g, z)


GROWS = NN // NS


def _sc_layer_body(src_hbm, dst_hbm, w_hbm, g_hbm, z_hbm, out_hbm,
                   si_v, di_v, w_v, rows_v, acc_sh, g_sh,
                   semz, semi, semg, sems):
    c = lax.axis_index("c")
    s = lax.axis_index("s")
    wid = s * NC + c

    zcp = pltpu.async_copy(z_hbm.at[pl.ds(s * SLICE, SLICE)],
                           acc_sh.at[pl.ds(s * SLICE, SLICE)], semz)
    gcp = pltpu.async_copy(g_hbm.at[pl.ds(s * GROWS, GROWS)],
                           g_sh.at[pl.ds(s * GROWS, GROWS)], semz)
    icps = [
        pltpu.async_copy(src_hbm.at[pl.ds(wid * ROWS_PW, ROWS_PW)], si_v, semi),
        pltpu.async_copy(dst_hbm.at[pl.ds(wid * ROWS_PW, ROWS_PW)], di_v, semi),
        pltpu.async_copy(w_hbm.at[pl.ds(wid * EPW, EPW)], w_v, semi),
    ]
    for cp in icps:
        cp.wait()

    def issue_gathers(t):
        slot = rows_v.at[t % NBUF]
        return [
            pltpu.async_copy(g_sh.at[si_v.at[t * RPC + j]],
                             slot.at[pl.ds(j * 128, 128)], semg)
            for j in range(RPC)
        ]

    zcp.wait()
    gcp.wait()
    plsc.subcore_barrier()
    gd = {0: issue_gathers(0), 1: issue_gathers(1)}

    sd = {}
    for t in range(NCHK):
        if t >= 2:
            for cp in sd[t - 2]:
                cp.wait()
        if t + 2 < NCHK:
            gd[t + 2] = issue_gathers(t + 2)
        for cp in gd[t]:
            cp.wait()
        slot = t % NBUF
        base_e = t * CH

        @pl.loop(0, CH, step=16)
        def _scale(e0, slot=slot, base_e=base_e):
            w16 = w_v[pl.ds(base_e + e0, 16)]
            for u in range(16):
                spl = _splat(w16, u)
                e = e0 + u
                rows_v[slot, e, :] = rows_v[slot, e, :] * spl

        sd[t] = [
            pltpu.async_copy(rows_v.at[slot].at[pl.ds(j * 128, 128)],
                             acc_sh.at[di_v.at[t * RPC + j]], sems, add=True)
            for j in range(RPC)
        ]

    for cp in sd[NCHK - 2]:
        cp.wait()
    for cp in sd[NCHK - 1]:
        cp.wait()
    plsc.subcore_barrier()
    pltpu.sync_copy(acc_sh.at[pl.ds(s * SLICE, SLICE)],
                    out_hbm.at[c, pl.ds(s * SLICE, SLICE)])


def _scalar_accumulate_and_reduce(body_per_group, out_hbm,
                                  d_v, w_v, acc_v, red_v, out_v, red_sh, sem,
                                  extra_cps):
    c = lax.axis_index("c")
    s = lax.axis_index("s")
    wid = s * NC + c

    @pl.loop(0, NPAD, step=128)
    def _zero(i):
        for u in range(8):
            acc_v[pl.ds(i + u * 16, 16)] = jnp.zeros((16,), jnp.float32)

    for cp in extra_cps:
        cp.wait()

    @pl.loop(0, EPW, step=64)
    def _grp(e0):
        for u in range(4):
            sl = pl.ds(e0 + u * 16, 16)
            plsc.addupdate_scatter(acc_v, [d_v[sl]], body_per_group(sl))

    pltpu.sync_copy(acc_v, red_sh.at[s])
    plsc.subcore_barrier()
    rcps = [
        pltpu.async_copy(red_sh.at[r, pl.ds(s * SLICE, SLICE)],
                         red_v.at[r], sem)
        for r in range(NS)
    ]
    for cp in rcps:
        cp.wait()

    @pl.loop(0, SLICE, step=16)
    def _sum(i):
        sl = pl.ds(i, 16)
        acc16 = red_v[0, sl]
        for r in range(1, NS):
            acc16 = acc16 + red_v[r, sl]
        out_v[sl] = acc16

    pltpu.sync_copy(out_v, out_hbm.at[c, pl.ds(s * SLICE, SLICE)])


_SCALAR_SCRATCH = [
    pltpu.VMEM((EPW,), jnp.int32),
    pltpu.VMEM((EPW,), jnp.float32),
    pltpu.VMEM((NPAD,), jnp.float32),
    pltpu.VMEM((NS, SLICE), jnp.float32),
    pltpu.VMEM((SLICE,), jnp.float32),
    pltpu.VMEM_SHARED((NS, NPAD), jnp.float32),
    pltpu.SemaphoreType.DMA,
]


def _sc_deg_body(dst_hbm, w_hbm, out_hbm,
                 d_v, w_v, acc_v, red_v, out_v, red_sh, sem):
    c = lax.axis_index("c")
    s = lax.axis_index("s")
    wid = s * NC + c
    cps = [
        pltpu.async_copy(dst_hbm.at[pl.ds(wid * EPW, EPW)], d_v, sem),
        pltpu.async_copy(w_hbm.at[pl.ds(wid * EPW, EPW)], w_v, sem),
    ]
    _scalar_accumulate_and_reduce(
        lambda sl: w_v[sl], out_hbm,
        d_v, w_v, acc_v, red_v, out_v, red_sh, sem, cps)


@functools.cache
def _sc_deg_kernel():
    return functools.partial(
        pl.kernel,
        out_type=jax.ShapeDtypeStruct((NC, NPAD), jnp.float32),
        mesh=_sc_mesh(),
        compiler_params=_sc_params(),
        scratch_types=_SCALAR_SCRATCH,
    )(_sc_deg_body)


def _sc_deg(dstp, wp):
    return _sc_deg_kernel()(dstp, wp)


def _sc_conv1_body(src_hbm, dst_hbm, w_hbm, hh_hbm, out_hbm,
                   src_v, hh_v, d_v, w_v, acc_v, red_v, out_v, red_sh, sem):
    c = lax.axis_index("c")
    s = lax.axis_index("s")
    wid = s * NC + c
    cps = [
        pltpu.async_copy(src_hbm.at[pl.ds(wid * EPW, EPW)], src_v, sem),
        pltpu.async_copy(dst_hbm.at[pl.ds(wid * EPW, EPW)], d_v, sem),
        pltpu.async_copy(w_hbm.at[pl.ds(wid * EPW, EPW)], w_v, sem),
        pltpu.async_copy(hh_hbm, hh_v, sem),
    ]
    _scalar_accumulate_and_reduce(
        lambda sl: w_v[sl] * plsc.load_gather(hh_v, [src_v[sl]]), out_hbm,
        d_v, w_v, acc_v, red_v, out_v, red_sh, sem, cps)


@functools.cache
def _sc_conv1_kernel():
    return functools.partial(
        pl.kernel,
        out_type=jax.ShapeDtypeStruct((NC, NPAD), jnp.float32),
        mesh=_sc_mesh(),
        compiler_params=_sc_params(),
        scratch_types=[pltpu.VMEM((EPW,), jnp.int32),
                       pltpu.VMEM((NPAD,), jnp.float32)] + _SCALAR_SCRATCH,
    )(_sc_conv1_body)


def _sc_conv1(srcp, dstp, wp, hh):
    return _sc_conv1_kernel()(srcp, dstp, wp, hh)

RB = 1000


def _dis_body(p_ref, o_ref):
    deg = p_ref[0] + p_ref[1] + 1.0
    o_ref[...] = jnp.where(deg > 0, lax.rsqrt(deg), 0.0)


def _tc_dis(degp):
    return pl.pallas_call(
        _dis_body,
        out_shape=jax.ShapeDtypeStruct((NPAD // 128, 128), jnp.float32),
    )(degp.reshape(NC, NPAD // 128, 128))


def _l1_body(x_ref, w_ref, dis_ref, o_ref):
    t = jnp.dot(x_ref[...], w_ref[...], preferred_element_type=jnp.float32)
    o_ref[...] = t * dis_ref[...]


def _tc_l1(x, W1, dis_col):
    return pl.pallas_call(
        _l1_body,
        grid=(NN // RB,),
        in_specs=[
            pl.BlockSpec((RB, DD), lambda i: (i, 0)),
            pl.BlockSpec((DD, HH), lambda i: (0, 0)),
            pl.BlockSpec((RB, 1), lambda i: (i, 0)),
        ],
        out_specs=pl.BlockSpec((RB, HH), lambda i: (i, 0)),
        out_shape=jax.ShapeDtypeStruct((NN, HH), jnp.float32),
    )(x, W1, dis_col)


def _mid_body(p_ref, g_ref, dis_ref, b_ref, w_ref, h_ref, gn_ref):
    dis = dis_ref[...]
    h = jnp.maximum(dis * (p_ref[0] + p_ref[1] + g_ref[...]) + b_ref[...], 0.0)
    h_ref[...] = h
    gn_ref[...] = dis * jnp.dot(h, w_ref[...],
                                preferred_element_type=jnp.float32)


def _tc_mid(p, g, dis_col, b_row, W_next):
    return pl.pallas_call(
        _mid_body,
        grid=(NN // RB,),
        in_specs=[
            pl.BlockSpec((NC, RB, HH), lambda i: (0, i, 0)),
            pl.BlockSpec((RB, HH), lambda i: (i, 0)),
            pl.BlockSpec((RB, 1), lambda i: (i, 0)),
            pl.BlockSpec((1, HH), lambda i: (0, 0)),
            pl.BlockSpec((HH, HH), lambda i: (0, 0)),
        ],
        out_specs=[
            pl.BlockSpec((RB, HH), lambda i: (i, 0)),
            pl.BlockSpec((RB, HH), lambda i: (i, 0)),
        ],
        out_shape=[
            jax.ShapeDtypeStruct((NN, HH), jnp.float32),
            jax.ShapeDtypeStruct((NN, HH), jnp.float32),
        ],
    )(p, g, dis_col, b_row, W_next)


def _tail_body(p_ref, g_ref, dis_ref, b_ref, h1_ref, h2_ref,
               wa_ref, wb_ref, wc_ref, lb_ref, wcl_ref,
               xl_ref, ghh_ref):
    dis = dis_ref[...]
    h3 = jnp.maximum(dis * (p_ref[0] + p_ref[1] + g_ref[...]) + b_ref[...], 0.0)
    xl = (jnp.dot(h1_ref[...], wa_ref[...], preferred_element_type=jnp.float32)
          + jnp.dot(h2_ref[...], wb_ref[...], preferred_element_type=jnp.float32)
          + jnp.dot(h3, wc_ref[...], preferred_element_type=jnp.float32)
          + lb_ref[...])
    xl = jnp.maximum(xl, 0.0)
    xl_ref[...] = xl
    ghh_ref[...] = dis * jnp.dot(xl, wcl_ref[...],
                                 preferred_element_type=jnp.float32)


def _tc_tail(p, g, dis_col, b_row, h1, h2, wa, wb, wc, lb, wcl):
    return pl.pallas_call(
        _tail_body,
        grid=(NN // RB,),
        in_specs=[
            pl.BlockSpec((NC, RB, HH), lambda i: (0, i, 0)),
            pl.BlockSpec((RB, HH), lambda i: (i, 0)),
            pl.BlockSpec((RB, 1), lambda i: (i, 0)),
            pl.BlockSpec((1, HH), lambda i: (0, 0)),
            pl.BlockSpec((RB, HH), lambda i: (i, 0)),
            pl.BlockSpec((RB, HH), lambda i: (i, 0)),
            pl.BlockSpec((HH, HH), lambda i: (0, 0)),
            pl.BlockSpec((HH, HH), lambda i: (0, 0)),
            pl.BlockSpec((HH, HH), lambda i: (0, 0)),
            pl.BlockSpec((1, HH), lambda i: (0, 0)),
            pl.BlockSpec((HH, 1), lambda i: (0, 0)),
        ],
        out_specs=[
            pl.BlockSpec((RB, HH), lambda i: (i, 0)),
            pl.BlockSpec((RB, 1), lambda i: (i, 0)),
        ],
        out_shape=[
            jax.ShapeDtypeStruct((NN, HH), jnp.float32),
            jax.ShapeDtypeStruct((NN, 1), jnp.float32),
        ],
    )(p, g, dis_col, b_row, h1, h2, wa, wb, wc, lb, wcl)


def _fin_body(q_ref, ghh_ref, dis_ref, bc_ref, xl_ref, w2_ref, b2_ref,
              ch_ref, val_ref):
    nrow = NPAD // 128
    c = dis_ref[...] * (q_ref[0] + q_ref[1] + ghh_ref[...]) + bc_ref[0, 0]
    flat = (lax.broadcasted_iota(jnp.int32, (nrow, 128), 0) * 128
            + lax.broadcasted_iota(jnp.int32, (nrow, 128), 1))
    valid = flat < NN
    c = jnp.where(valid, c, -jnp.inf)
    m = jnp.max(c)
    ex = jnp.exp(c - m)
    ch_ref[...] = ex / jnp.sum(ex)
    v = jnp.mean(xl_ref[...], axis=0, keepdims=True)
    val_ref[...] = jnp.dot(v, w2_ref[...],
                           preferred_element_type=jnp.float32) + b2_ref[...]


def _tc_fin(q, ghh_pad, dis2, bc, xl, lin2_W, lin2_b):
    return pl.pallas_call(
        _fin_body,
        out_shape=[
            jax.ShapeDtypeStruct((NPAD // 128, 128), jnp.float32),
            jax.ShapeDtypeStruct((1, 1), jnp.float32),
        ],
    )(q.reshape(NC, NPAD // 128, 128), ghh_pad.reshape(NPAD // 128, 128),
      dis2, bc.reshape(1, 1), xl, lin2_W, lin2_b.reshape(1, 1))


def kernel(x, edge_index, weight, W1, b1, W2, b2, W3, b3,
           lin1_W, lin1_b, Wc, bc, lin2_W, lin2_b):
    f32 = jnp.float32
    src = edge_index[0]
    dst = edge_index[1]
    w = weight.astype(f32)
    pad = EPAD - EE
    srcp = jnp.concatenate([src, jnp.zeros((pad,), src.dtype)])
    dstp = jnp.concatenate([dst, jnp.zeros((pad,), dst.dtype)])
    wp = jnp.concatenate([w, jnp.zeros((pad,), f32)])
    src2d = srcp.reshape(EPAD // 128, 128)
    dst2d = dstp.reshape(EPAD // 128, 128)
    zeros16 = jnp.zeros((NPAD, HH), f32)

    degp = _sc_deg(dstp, wp)
    dis2 = _tc_dis(degp)
    dis_col = dis2.reshape(NPAD, 1)[:NN]
    g1 = _tc_l1(x, W1, dis_col)
    p1 = _sc_layer(src2d, dst2d, wp, g1, zeros16)
    h1, g2 = _tc_mid(p1, g1, dis_col, b1.reshape(1, HH), W2)
    p2 = _sc_layer(src2d, dst2d, wp, g2, zeros16)
    h2, g3 = _tc_mid(p2, g2, dis_col, b2.reshape(1, HH), W3)
    p3 = _sc_layer(src2d, dst2d, wp, g3, zeros16)

    wa = lin1_W[:HH]
    wb = lin1_W[HH:2 * HH]
    wc = lin1_W[2 * HH:]
    xl, ghh = _tc_tail(p3, g3, dis_col, b3.reshape(1, HH), h1, h2,
                       wa, wb, wc, lin1_b.reshape(1, HH), Wc)

    ghh_pad = jnp.pad(ghh.reshape(NN), (0, NPAD - NN))
    q = _sc_conv1(srcp, dstp, wp, ghh_pad)
    ch2, value = _tc_fin(q, ghh_pad, dis2, bc, xl, lin2_W, lin2_b)
    choice = ch2.reshape(NPAD)[:NN]
    return (choice, value)
